# baseline (speedup 1.0000x reference)
import functools

import jax
import jax.numpy as jnp
from jax import lax
from jax.experimental import pallas as pl
from jax.experimental.pallas import tpu as pltpu
from jax.experimental.pallas import tpu_sc as plsc

TOPK = 8
BN_EPS = 1e-5
NEG_INF = float("-inf")


def _mlp_stats_body(x_ref, w1_ref, b1_ref, h_ref, stats_ref):
    i = pl.program_id(0)
    h = jnp.dot(x_ref[...], w1_ref[...], preferred_element_type=jnp.float32)
    h = jnp.maximum(h + b1_ref[...][None, :], 0.0)
    h_ref[...] = h

    s = jnp.sum(h, axis=0)
    ss = jnp.sum(h * h, axis=0)
    upd = jnp.concatenate(
        [s[None, :], ss[None, :], jnp.zeros((6, s.shape[0]), jnp.float32)], axis=0
    )

    @pl.when(i == 0)
    def _():
        stats_ref[...] = jnp.zeros_like(stats_ref)

    stats_ref[...] += upd


def _logits_body(nrows, h_ref, stats_ref, gamma_ref, beta_ref, w2_ref, b2_ref,
                 out_ref):
    inv_n = 1.0 / nrows
    mean = stats_ref[0, :] * inv_n
    var = stats_ref[1, :] * inv_n - mean * mean
    rstd = 1.0 / jnp.sqrt(var + BN_EPS)
    scale = gamma_ref[...] * rstd
    shift = beta_ref[...] - mean * scale

    hn = h_ref[...] * scale[None, :] + shift[None, :]
    logits = jnp.dot(hn, w2_ref[...], preferred_element_type=jnp.float32)
    logits = logits + b2_ref[...]
    out_ref[...] = logits.T[None]


def _sc_topk_softmax(logits_slabs, B, E):
    info = plsc.get_sparse_core_info()
    NC, NS, L = info.num_cores, info.num_subcores, info.num_lanes
    NW = NC * NS
    rpw = B // NW
    ngroups = rpw // L
    slab = E * rpw

    mesh = plsc.VectorSubcoreMesh(core_axis_name="c", subcore_axis_name="s")

    def _insert(regs, v):
        new = v
        out = []
        for j in range(TOPK):
            hi = jnp.maximum(regs[j], new)
            lo = jnp.minimum(regs[j], new)
            out.append(hi)
            new = lo
        return tuple(out)

    @functools.partial(
        pl.kernel,
        mesh=mesh,
        out_type=jax.ShapeDtypeStruct((NW * slab,), jnp.float32),
        scratch_types=[
            pltpu.VMEM((slab,), jnp.float32),
            pltpu.VMEM((slab,), jnp.float32),
        ],
    )
    def topk_kernel(logits_hbm, out_hbm, buf, obuf):
        wid = lax.axis_index("s") * NC + lax.axis_index("c")
        base = wid * slab
        pltpu.sync_copy(logits_hbm.at[pl.ds(base, slab)], buf)
        neg = jnp.full((L,), NEG_INF, jnp.float32)

        def group(g, _):
            off = g * L

            def p1(e2, carry):
                a = carry[:TOPK]
                b = carry[TOPK:]
                va = buf[pl.ds((2 * e2) * rpw + off, L)]
                vb = buf[pl.ds((2 * e2 + 1) * rpw + off, L)]
                return _insert(a, va) + _insert(b, vb)

            carry = lax.fori_loop(0, E // 2, p1, (neg,) * (2 * TOPK))
            a = carry[:TOPK]
            b = carry[TOPK:]
            for j in range(TOPK):
                a = _insert(a, b[j])
            m0 = a[0]
            t = a[TOPK - 1]

            def sm(e, den):
                v = buf[pl.ds(e * rpw + off, L)]
                p_ = jnp.where(v >= t, jnp.exp(v - m0), 0.0)
                obuf[pl.ds(e * rpw + off, L)] = p_
                return den + p_

            den = lax.fori_loop(0, E, sm, jnp.zeros((L,), jnp.float32))
            r = 1.0 / den

            def st(e, _):
                idx = pl.ds(e * rpw + off, L)
                obuf[idx] = obuf[idx] * r
                return 0

            lax.fori_loop(0, E, st, 0)
            return 0

        lax.fori_loop(0, ngroups, group, 0)
        pltpu.sync_copy(obuf, out_hbm.at[pl.ds(base, slab)])

    return topk_kernel(logits_slabs)


def kernel(x, W1, b1, gamma, beta, W2, b2):
    B, D = x.shape
    H = W1.shape[1]
    E = W2.shape[1]
    NW = 32
    rpw = B // NW

    TB_A = 256
    grid_a = B // TB_A
    h, stats = pl.pallas_call(
        _mlp_stats_body,
        grid=(grid_a,),
        in_specs=[
            pl.BlockSpec((TB_A, D), lambda i: (i, 0)),
            pl.BlockSpec((D, H), lambda i: (0, 0)),
            pl.BlockSpec((H,), lambda i: (0,)),
        ],
        out_specs=[
            pl.BlockSpec((TB_A, H), lambda i: (i, 0)),
            pl.BlockSpec((8, H), lambda i: (0, 0)),
        ],
        out_shape=[
            jax.ShapeDtypeStruct((B, H), jnp.float32),
            jax.ShapeDtypeStruct((8, H), jnp.float32),
        ],
        compiler_params=pltpu.CompilerParams(
            dimension_semantics=("arbitrary",),
        ),
    )(x, W1, b1)

    NCHUNK = 2
    BC = B // NCHUNK
    rpw_c = BC // NW

    outs = []
    for c in range(NCHUNK):
        hc = lax.slice_in_dim(h, c * BC, (c + 1) * BC, axis=0)
        logits_slabs = pl.pallas_call(
            functools.partial(_logits_body, float(B)),
            grid=(NW,),
            in_specs=[
                pl.BlockSpec((rpw_c, H), lambda i: (i, 0)),
                pl.BlockSpec((8, H), lambda i: (0, 0)),
                pl.BlockSpec((H,), lambda i: (0,)),
                pl.BlockSpec((H,), lambda i: (0,)),
                pl.BlockSpec((H, E), lambda i: (0, 0)),
                pl.BlockSpec((1, E), lambda i: (0, 0)),
            ],
            out_specs=pl.BlockSpec((1, E, rpw_c), lambda i: (i, 0, 0)),
            out_shape=jax.ShapeDtypeStruct((NW, E, rpw_c), jnp.float32),
            compiler_params=pltpu.CompilerParams(
                dimension_semantics=("arbitrary",),
            ),
        )(hc, stats, gamma, beta, W2, b2[None, :])

        probs_slabs = _sc_topk_softmax(
            logits_slabs.reshape(NW * E * rpw_c), BC, E)
        outs.append(
            probs_slabs.reshape(NW, E, rpw_c).transpose(0, 2, 1).reshape(BC, E))
    return jnp.concatenate(outs, axis=0)


# speedup vs baseline: 1.1357x; 1.1357x over previous
import functools

import jax
import jax.numpy as jnp
from jax import lax
from jax.experimental import pallas as pl
from jax.experimental.pallas import tpu as pltpu
from jax.experimental.pallas import tpu_sc as plsc

TOPK = 8
BN_EPS = 1e-5
NEG_INF = float("-inf")


def _mlp_stats_body(x_ref, w1_ref, b1_ref, h_ref, stats_ref):
    i = pl.program_id(0)
    h = jnp.dot(x_ref[...], w1_ref[...], preferred_element_type=jnp.float32)
    h = jnp.maximum(h + b1_ref[...][None, :], 0.0)
    h_ref[...] = h

    s = jnp.sum(h, axis=0)
    ss = jnp.sum(h * h, axis=0)
    upd = jnp.concatenate(
        [s[None, :], ss[None, :], jnp.zeros((6, s.shape[0]), jnp.float32)], axis=0
    )

    @pl.when(i == 0)
    def _():
        stats_ref[...] = jnp.zeros_like(stats_ref)

    stats_ref[...] += upd


def _logits_body(nrows, h_ref, stats_ref, gamma_ref, beta_ref, w2_ref, b2_ref,
                 out_ref):
    inv_n = 1.0 / nrows
    mean = stats_ref[0, :] * inv_n
    var = stats_ref[1, :] * inv_n - mean * mean
    rstd = 1.0 / jnp.sqrt(var + BN_EPS)
    scale = gamma_ref[...] * rstd
    shift = beta_ref[...] - mean * scale

    hn = h_ref[...] * scale[None, :] + shift[None, :]
    logits = jnp.dot(hn, w2_ref[...], preferred_element_type=jnp.float32)
    logits = logits + b2_ref[...]
    out_ref[...] = logits.T[None]


def _sc_topk_softmax(logits_slabs, B, E):
    info = plsc.get_sparse_core_info()
    NC, NS, L = info.num_cores, info.num_subcores, info.num_lanes
    NW = NC * NS
    rpw = B // NW
    ngroups = rpw // L
    slab = E * rpw

    mesh = plsc.VectorSubcoreMesh(core_axis_name="c", subcore_axis_name="s")

    def _insert(regs, v):
        new = v
        out = []
        for j in range(TOPK):
            hi = jnp.maximum(regs[j], new)
            lo = jnp.minimum(regs[j], new)
            out.append(hi)
            new = lo
        return tuple(out)

    @functools.partial(
        pl.kernel,
        mesh=mesh,
        out_type=jax.ShapeDtypeStruct((NW * slab,), jnp.float32),
        scratch_types=[
            pltpu.VMEM((slab,), jnp.float32),
            pltpu.VMEM((slab,), jnp.float32),
        ],
    )
    def topk_kernel(logits_hbm, out_hbm, buf, obuf):
        wid = lax.axis_index("s") * NC + lax.axis_index("c")
        base = wid * slab
        pltpu.sync_copy(logits_hbm.at[pl.ds(base, slab)], buf)
        neg = jnp.full((L,), NEG_INF, jnp.float32)

        def group(g, _):
            off = g * L

            def p1(e2, carry):
                a = carry[:TOPK]
                b = carry[TOPK:]
                va = buf[pl.ds((2 * e2) * rpw + off, L)]
                vb = buf[pl.ds((2 * e2 + 1) * rpw + off, L)]
                return _insert(a, va) + _insert(b, vb)

            carry = lax.fori_loop(0, E // 2, p1, (neg,) * (2 * TOPK))
            a = carry[:TOPK]
            b = carry[TOPK:]
            for j in range(TOPK):
                a = _insert(a, b[j])
            m0 = a[0]
            t = a[TOPK - 1]

            den = jnp.exp(a[0] - m0)
            for j in range(1, TOPK):
                den = den + jnp.exp(a[j] - m0)
            r = 1.0 / den

            def st(e, _):
                v = buf[pl.ds(e * rpw + off, L)]
                p_ = jnp.where(v >= t, jnp.exp(v - m0) * r, 0.0)
                obuf[pl.ds(e * rpw + off, L)] = p_
                return 0

            lax.fori_loop(0, E, st, 0)
            return 0

        lax.fori_loop(0, ngroups, group, 0)
        pltpu.sync_copy(obuf, out_hbm.at[pl.ds(base, slab)])

    return topk_kernel(logits_slabs)


def kernel(x, W1, b1, gamma, beta, W2, b2):
    B, D = x.shape
    H = W1.shape[1]
    E = W2.shape[1]
    NW = 32
    rpw = B // NW

    TB_A = 256
    grid_a = B // TB_A
    h, stats = pl.pallas_call(
        _mlp_stats_body,
        grid=(grid_a,),
        in_specs=[
            pl.BlockSpec((TB_A, D), lambda i: (i, 0)),
            pl.BlockSpec((D, H), lambda i: (0, 0)),
            pl.BlockSpec((H,), lambda i: (0,)),
        ],
        out_specs=[
            pl.BlockSpec((TB_A, H), lambda i: (i, 0)),
            pl.BlockSpec((8, H), lambda i: (0, 0)),
        ],
        out_shape=[
            jax.ShapeDtypeStruct((B, H), jnp.float32),
            jax.ShapeDtypeStruct((8, H), jnp.float32),
        ],
        compiler_params=pltpu.CompilerParams(
            dimension_semantics=("arbitrary",),
        ),
    )(x, W1, b1)

    logits_slabs = pl.pallas_call(
        functools.partial(_logits_body, float(B)),
        grid=(NW,),
        in_specs=[
            pl.BlockSpec((rpw, H), lambda i: (i, 0)),
            pl.BlockSpec((8, H), lambda i: (0, 0)),
            pl.BlockSpec((H,), lambda i: (0,)),
            pl.BlockSpec((H,), lambda i: (0,)),
            pl.BlockSpec((H, E), lambda i: (0, 0)),
            pl.BlockSpec((1, E), lambda i: (0, 0)),
        ],
        out_specs=pl.BlockSpec((1, E, rpw), lambda i: (i, 0, 0)),
        out_shape=jax.ShapeDtypeStruct((NW, E, rpw), jnp.float32),
        compiler_params=pltpu.CompilerParams(
            dimension_semantics=("arbitrary",),
        ),
    )(h, stats, gamma, beta, W2, b2[None, :])

    probs_slabs = _sc_topk_softmax(logits_slabs.reshape(NW * E * rpw), B, E)
    return probs_slabs.reshape(NW, E, rpw).transpose(0, 2, 1).reshape(B, E)


# speedup vs baseline: 1.2223x; 1.0762x over previous
import functools

import jax
import jax.numpy as jnp
from jax import lax
from jax.experimental import pallas as pl
from jax.experimental.pallas import tpu as pltpu
from jax.experimental.pallas import tpu_sc as plsc

TOPK = 8
BN_EPS = 1e-5
NEG_INF = float("-inf")


def _mlp_stats_body(x_ref, w1_ref, b1_ref, h_ref, stats_ref):
    i = pl.program_id(0)
    h = jnp.dot(x_ref[...], w1_ref[...], preferred_element_type=jnp.float32)
    h = jnp.maximum(h + b1_ref[...][None, :], 0.0)
    h_ref[...] = h

    s = jnp.sum(h, axis=0)
    ss = jnp.sum(h * h, axis=0)
    upd = jnp.concatenate(
        [s[None, :], ss[None, :], jnp.zeros((6, s.shape[0]), jnp.float32)], axis=0
    )

    @pl.when(i == 0)
    def _():
        stats_ref[...] = jnp.zeros_like(stats_ref)

    stats_ref[...] += upd


def _logits_body(nrows, h_ref, stats_ref, gamma_ref, beta_ref, w2_ref, b2_ref,
                 out_ref):
    inv_n = 1.0 / nrows
    mean = stats_ref[0, :] * inv_n
    var = stats_ref[1, :] * inv_n - mean * mean
    rstd = 1.0 / jnp.sqrt(var + BN_EPS)
    scale = gamma_ref[...] * rstd
    shift = beta_ref[...] - mean * scale

    hn = h_ref[...] * scale[None, :] + shift[None, :]
    logits = jnp.dot(hn, w2_ref[...], preferred_element_type=jnp.float32)
    logits = logits + b2_ref[...]
    out_ref[...] = logits.T[None]


def _sc_topk_softmax(logits_slabs, B, E):
    info = plsc.get_sparse_core_info()
    NC, NS, L = info.num_cores, info.num_subcores, info.num_lanes
    NW = NC * NS
    rpw = B // NW
    ngroups = rpw // L
    slab = E * rpw

    mesh = plsc.VectorSubcoreMesh(core_axis_name="c", subcore_axis_name="s")

    def _insert(regs, v):
        new = v
        out = []
        for j in range(TOPK):
            hi = jnp.maximum(regs[j], new)
            lo = jnp.minimum(regs[j], new)
            out.append(hi)
            new = lo
        return tuple(out)

    @functools.partial(
        pl.kernel,
        mesh=mesh,
        out_type=jax.ShapeDtypeStruct((NW * slab,), jnp.float32),
        scratch_types=[
            pltpu.VMEM((slab,), jnp.float32),
            pltpu.VMEM((slab,), jnp.float32),
        ],
    )
    def topk_kernel(logits_hbm, out_hbm, buf, obuf):
        wid = lax.axis_index("s") * NC + lax.axis_index("c")
        base = wid * slab
        pltpu.sync_copy(logits_hbm.at[pl.ds(base, slab)], buf)
        neg = jnp.full((L,), NEG_INF, jnp.float32)

        def group(g, _):
            off = g * L

            def p1(e2, carry):
                a = carry[:TOPK]
                b = carry[TOPK:]
                va = buf[pl.ds((2 * e2) * rpw + off, L)]
                vb = buf[pl.ds((2 * e2 + 1) * rpw + off, L)]
                return _insert(a, va) + _insert(b, vb)

            carry = lax.fori_loop(0, E // 2, p1, (neg,) * (2 * TOPK))
            a = carry[:TOPK]
            b = carry[TOPK:]
            for j in range(TOPK):
                a = _insert(a, b[j])
            m0 = a[0]
            t = a[TOPK - 1]

            den = jnp.exp(a[0] - m0)
            for j in range(1, TOPK):
                den = den + jnp.exp(a[j] - m0)
            r = 1.0 / den

            def st(e, _):
                v = buf[pl.ds(e * rpw + off, L)]
                p_ = jnp.where(v >= t, jnp.exp(v - m0) * r, 0.0)
                obuf[pl.ds(e * rpw + off, L)] = p_
                return 0

            lax.fori_loop(0, E, st, 0)
            return 0

        lax.fori_loop(0, ngroups, group, 0)
        pltpu.sync_copy(obuf, out_hbm.at[pl.ds(base, slab)])

    return topk_kernel(logits_slabs)


def kernel(x, W1, b1, gamma, beta, W2, b2):
    B, D = x.shape
    H = W1.shape[1]
    E = W2.shape[1]
    NW = 32
    rpw = B // NW

    TB_A = 512
    grid_a = B // TB_A
    h, stats = pl.pallas_call(
        _mlp_stats_body,
        grid=(grid_a,),
        in_specs=[
            pl.BlockSpec((TB_A, D), lambda i: (i, 0)),
            pl.BlockSpec((D, H), lambda i: (0, 0)),
            pl.BlockSpec((H,), lambda i: (0,)),
        ],
        out_specs=[
            pl.BlockSpec((TB_A, H), lambda i: (i, 0)),
            pl.BlockSpec((8, H), lambda i: (0, 0)),
        ],
        out_shape=[
            jax.ShapeDtypeStruct((B, H), jnp.float32),
            jax.ShapeDtypeStruct((8, H), jnp.float32),
        ],
        compiler_params=pltpu.CompilerParams(
            dimension_semantics=("arbitrary",),
        ),
    )(x, W1, b1)

    logits_slabs = pl.pallas_call(
        functools.partial(_logits_body, float(B)),
        grid=(NW,),
        in_specs=[
            pl.BlockSpec((rpw, H), lambda i: (i, 0)),
            pl.BlockSpec((8, H), lambda i: (0, 0)),
            pl.BlockSpec((H,), lambda i: (0,)),
            pl.BlockSpec((H,), lambda i: (0,)),
            pl.BlockSpec((H, E), lambda i: (0, 0)),
            pl.BlockSpec((1, E), lambda i: (0, 0)),
        ],
        out_specs=pl.BlockSpec((1, E, rpw), lambda i: (i, 0, 0)),
        out_shape=jax.ShapeDtypeStruct((NW, E, rpw), jnp.float32),
        compiler_params=pltpu.CompilerParams(
            dimension_semantics=("arbitrary",),
        ),
    )(h, stats, gamma, beta, W2, b2[None, :])

    probs_slabs = _sc_topk_softmax(logits_slabs.reshape(NW * E * rpw), B, E)
    return probs_slabs.reshape(NW, E, rpw).transpose(0, 2, 1).reshape(B, E)


# speedup vs baseline: 1.2469x; 1.0201x over previous
import functools

import jax
import jax.numpy as jnp
from jax import lax
from jax.experimental import pallas as pl
from jax.experimental.pallas import tpu as pltpu
from jax.experimental.pallas import tpu_sc as plsc

TOPK = 8
BN_EPS = 1e-5
NEG_INF = float("-inf")


def _mlp_stats_body(x_ref, w1_ref, b1_ref, h_ref, stats_ref):
    i = pl.program_id(0)
    h = jnp.dot(x_ref[...], w1_ref[...], preferred_element_type=jnp.float32)
    h = jnp.maximum(h + b1_ref[...][None, :], 0.0)
    h_ref[...] = h

    s = jnp.sum(h, axis=0)
    ss = jnp.sum(h * h, axis=0)
    upd = jnp.concatenate(
        [s[None, :], ss[None, :], jnp.zeros((6, s.shape[0]), jnp.float32)], axis=0
    )

    @pl.when(i == 0)
    def _():
        stats_ref[...] = jnp.zeros_like(stats_ref)

    stats_ref[...] += upd


def _logits_body(nrows, h_ref, stats_ref, gamma_ref, beta_ref, w2_ref, b2_ref,
                 out_ref):
    inv_n = 1.0 / nrows
    mean = stats_ref[0, :] * inv_n
    var = stats_ref[1, :] * inv_n - mean * mean
    rstd = 1.0 / jnp.sqrt(var + BN_EPS)
    scale = gamma_ref[...] * rstd
    shift = beta_ref[...] - mean * scale

    hn = h_ref[...] * scale[None, :] + shift[None, :]
    logits = jnp.dot(hn, w2_ref[...], preferred_element_type=jnp.float32)
    logits = logits + b2_ref[...]
    out_ref[...] = logits.T[None]


def _sc_topk_softmax(logits_slabs, B, E):
    info = plsc.get_sparse_core_info()
    NC, NS, L = info.num_cores, info.num_subcores, info.num_lanes
    NW = NC * NS
    rpw = B // NW
    ngroups = rpw // L
    slab = E * rpw

    mesh = plsc.VectorSubcoreMesh(core_axis_name="c", subcore_axis_name="s")

    def _insert(regs, v):
        new = v
        out = []
        for j in range(TOPK):
            hi = jnp.maximum(regs[j], new)
            lo = jnp.minimum(regs[j], new)
            out.append(hi)
            new = lo
        return tuple(out)

    @functools.partial(
        pl.kernel,
        mesh=mesh,
        out_type=jax.ShapeDtypeStruct((NW * slab,), jnp.float32),
        scratch_types=[
            pltpu.VMEM((slab,), jnp.float32),
            pltpu.VMEM((slab,), jnp.float32),
        ],
    )
    def topk_kernel(logits_hbm, out_hbm, buf, obuf):
        wid = lax.axis_index("s") * NC + lax.axis_index("c")
        base = wid * slab
        pltpu.sync_copy(logits_hbm.at[pl.ds(base, slab)], buf)
        neg = jnp.full((L,), NEG_INF, jnp.float32)

        def group(g, _):
            off = g * L

            def p1(e2, carry):
                a = carry[:TOPK]
                b = carry[TOPK:]
                va = buf[pl.ds((2 * e2) * rpw + off, L)]
                vb = buf[pl.ds((2 * e2 + 1) * rpw + off, L)]
                return _insert(a, va) + _insert(b, vb)

            carry = lax.fori_loop(0, E // 2, p1, (neg,) * (2 * TOPK))
            a = carry[:TOPK]
            b = carry[TOPK:]
            for j in range(TOPK):
                a = _insert(a, b[j])
            m0 = a[0]
            t = a[TOPK - 1]

            den = jnp.exp(a[0] - m0)
            for j in range(1, TOPK):
                den = den + jnp.exp(a[j] - m0)
            r = 1.0 / den

            def st(e, _):
                v = buf[pl.ds(e * rpw + off, L)]
                p_ = jnp.where(v >= t, jnp.exp(v - m0) * r, 0.0)
                obuf[pl.ds(e * rpw + off, L)] = p_
                return 0

            lax.fori_loop(0, E, st, 0)
            return 0

        lax.fori_loop(0, ngroups, group, 0)
        pltpu.sync_copy(obuf, out_hbm.at[pl.ds(base, slab)])

    return topk_kernel(logits_slabs)


def kernel(x, W1, b1, gamma, beta, W2, b2):
    B, D = x.shape
    H = W1.shape[1]
    E = W2.shape[1]
    NW = 32
    rpw = B // NW

    TB_A = 1024
    grid_a = B // TB_A
    h, stats = pl.pallas_call(
        _mlp_stats_body,
        grid=(grid_a,),
        in_specs=[
            pl.BlockSpec((TB_A, D), lambda i: (i, 0)),
            pl.BlockSpec((D, H), lambda i: (0, 0)),
            pl.BlockSpec((H,), lambda i: (0,)),
        ],
        out_specs=[
            pl.BlockSpec((TB_A, H), lambda i: (i, 0)),
            pl.BlockSpec((8, H), lambda i: (0, 0)),
        ],
        out_shape=[
            jax.ShapeDtypeStruct((B, H), jnp.float32),
            jax.ShapeDtypeStruct((8, H), jnp.float32),
        ],
        compiler_params=pltpu.CompilerParams(
            dimension_semantics=("arbitrary",),
        ),
    )(x, W1, b1)

    logits_slabs = pl.pallas_call(
        functools.partial(_logits_body, float(B)),
        grid=(NW,),
        in_specs=[
            pl.BlockSpec((rpw, H), lambda i: (i, 0)),
            pl.BlockSpec((8, H), lambda i: (0, 0)),
            pl.BlockSpec((H,), lambda i: (0,)),
            pl.BlockSpec((H,), lambda i: (0,)),
            pl.BlockSpec((H, E), lambda i: (0, 0)),
            pl.BlockSpec((1, E), lambda i: (0, 0)),
        ],
        out_specs=pl.BlockSpec((1, E, rpw), lambda i: (i, 0, 0)),
        out_shape=jax.ShapeDtypeStruct((NW, E, rpw), jnp.float32),
        compiler_params=pltpu.CompilerParams(
            dimension_semantics=("arbitrary",),
        ),
    )(h, stats, gamma, beta, W2, b2[None, :])

    probs_slabs = _sc_topk_softmax(logits_slabs.reshape(NW * E * rpw), B, E)
    return probs_slabs.reshape(NW, E, rpw).transpose(0, 2, 1).reshape(B, E)
